# Initial kernel scaffold; baseline (speedup 1.0000x reference)
import jax
import jax.numpy as jnp
from jax.experimental import pallas as pl


def kernel(x, adj, W1, b1, W2, b2):
    raise NotImplementedError("write your pallas kernel here")


# speedup vs baseline: 5.3782x; 5.3782x over previous
import functools

import jax
import jax.numpy as jnp
from jax import lax
from jax.experimental import pallas as pl
from jax.experimental.pallas import tpu as pltpu
from jax.experimental.pallas import tpu_sc as plsc

N = 10000
E = 320000
D = 128

NC = 2
NS = 16
NW = NC * NS
CHUNK = 128
EP = 327680
SCHUNK = EP // (NS * CHUNK)
NH = 5000
NHPAD = 5120
SRPT = NHPAD // NS

_mesh = plsc.VectorSubcoreMesh(core_axis_name="c", subcore_axis_name="s")


def _zero_slice(buf, acc, base, nrows):
    for k in range(nrows // CHUNK):
        pltpu.sync_copy(buf, acc.at[pl.ds(base + k * CHUNK, CHUNK)])
    rem = nrows % CHUNK
    if rem:
        pltpu.sync_copy(buf.at[pl.ds(0, rem)],
                        acc.at[pl.ds(base + nrows - rem, rem)])


@functools.partial(
    pl.kernel,
    out_type=jax.ShapeDtypeStruct((NC, NHPAD, D), jnp.float32),
    mesh=_mesh,
    scratch_types=[
        pltpu.VMEM((SCHUNK, CHUNK), jnp.int32),
        pltpu.VMEM((CHUNK, D), jnp.float32),
        pltpu.VMEM_SHARED((NHPAD, D), jnp.float32),
    ],
)
def _deg_kernel(dst3, out, dst_v, buf, dacc):
    c = lax.axis_index("c")
    s = lax.axis_index("s")
    pltpu.sync_copy(dst3.at[s], dst_v)

    def fill(val):
        def row(i, carry):
            for k in range(D // 16):
                buf[i, pl.ds(k * 16, 16)] = jnp.full((16,), val, jnp.float32)
            return carry
        lax.fori_loop(0, CHUNK, row, 0)

    fill(0.0)
    base = s * SRPT
    _zero_slice(buf, dacc, base, SRPT)

    lo = c * NH

    def remap(j, carry):
        for k in range(CHUNK // 16):
            v = dst_v[j, pl.ds(k * 16, 16)]
            lc = v - lo
            ok = (lc >= 0) & (lc < NH)
            dst_v[j, pl.ds(k * 16, 16)] = jnp.where(ok, lc, NH)
        return carry
    lax.fori_loop(0, SCHUNK, remap, 0)

    fill(1.0)
    plsc.subcore_barrier()

    def chunk(j, carry):
        pltpu.sync_copy(buf, dacc.at[dst_v.at[j]], add=True)
        return carry
    lax.fori_loop(0, SCHUNK, chunk, 0)
    plsc.subcore_barrier()
    pltpu.sync_copy(dacc.at[pl.ds(base, SRPT)], out.at[c, pl.ds(base, SRPT)])


@functools.partial(
    pl.kernel,
    out_type=jax.ShapeDtypeStruct((NC, NHPAD, D), jnp.float32),
    mesh=_mesh,
    scratch_types=[
        pltpu.VMEM((SCHUNK, CHUNK), jnp.int32),
        pltpu.VMEM((SCHUNK, CHUNK), jnp.int32),
        pltpu.VMEM((CHUNK, D), jnp.float32),
        pltpu.VMEM((CHUNK, D), jnp.float32),
        pltpu.SemaphoreType.DMA,
        pltpu.SemaphoreType.DMA,
        pltpu.VMEM_SHARED((NHPAD, D), jnp.float32),
    ],
)
def _scatter_kernel(g, src3, dst3, out, src_v, dst_v, buf0, buf1,
                    sem0, sem1, acc):
    c = lax.axis_index("c")
    s = lax.axis_index("s")
    pltpu.sync_copy(src3.at[s], src_v)
    pltpu.sync_copy(dst3.at[s], dst_v)
    def zrow(i, carry):
        for k in range(D // 16):
            buf0[i, pl.ds(k * 16, 16)] = jnp.zeros((16,), jnp.float32)
        return carry
    lax.fori_loop(0, CHUNK, zrow, 0)
    base = s * SRPT
    _zero_slice(buf0, acc, base, SRPT)

    lo = c * NH

    def remap(j, carry):
        for k in range(CHUNK // 16):
            v = dst_v[j, pl.ds(k * 16, 16)]
            lc = v - lo
            ok = (lc >= 0) & (lc < NH)
            dst_v[j, pl.ds(k * 16, 16)] = jnp.where(ok, lc, NH)
        return carry
    lax.fori_loop(0, SCHUNK, remap, 0)
    plsc.subcore_barrier()

    def fire(j, buf, sem):
        pltpu.async_copy(g.at[src_v.at[j]], buf, sem)

    def wait(buf, sem):
        pltpu.make_async_copy(g.at[pl.ds(0, CHUNK)], buf, sem).wait()

    fire(0, buf0, sem0)

    def pair(i, carry):
        j0 = 2 * i
        fire(j0 + 1, buf1, sem1)
        wait(buf0, sem0)
        pltpu.sync_copy(buf0, acc.at[dst_v.at[j0]], add=True)

        @pl.when(j0 + 2 < SCHUNK)
        def _():
            fire(j0 + 2, buf0, sem0)

        wait(buf1, sem1)
        pltpu.sync_copy(buf1, acc.at[dst_v.at[j0 + 1]], add=True)
        return carry

    lax.fori_loop(0, SCHUNK // 2, pair, 0)
    plsc.subcore_barrier()
    pltpu.sync_copy(acc.at[pl.ds(base, SRPT)], out.at[c, pl.ds(base, SRPT)])


R = 1000


def _dinv_of(dp_ref):
    deg = dp_ref[0, :, 0] + 1.0
    return lax.rsqrt(deg)


def _tc1_body(x_ref, w_ref, dp_ref, g_ref, t_ref):
    dinv = _dinv_of(dp_ref)
    t = jnp.dot(x_ref[...], w_ref[...], preferred_element_type=jnp.float32)
    t_ref[...] = t
    g_ref[...] = t * dinv[:, None]


def _tc2_body(s_ref, dp_ref, t1_ref, w_ref, b_ref, t2_ref, g2_ref):
    dinv = _dinv_of(dp_ref)
    h = (s_ref[0] * dinv[:, None]
         + t1_ref[...] * (dinv * dinv)[:, None] + b_ref[...])
    t2 = jnp.dot(h, w_ref[...], preferred_element_type=jnp.float32)
    t2_ref[...] = t2
    g2_ref[...] = t2 * dinv[:, None]


def _tc3_body(s_ref, dp_ref, t2_ref, b_ref, out_ref):
    dinv = _dinv_of(dp_ref)
    out_ref[...] = (s_ref[0] * dinv[:, None]
                    + t2_ref[...] * (dinv * dinv)[:, None] + b_ref[...])


_row_spec = pl.BlockSpec((R, D), lambda i: (i, 0))
_w_spec = pl.BlockSpec((D, D), lambda i: (0, 0))
_s_spec = pl.BlockSpec((1, R, D), lambda i: (i // (NH // R), i % (NH // R), 0))
_b_spec = pl.BlockSpec((1, D), lambda i: (0, 0))

_tc1 = pl.pallas_call(
    _tc1_body,
    grid=(N // R,),
    in_specs=[_row_spec, _w_spec, _s_spec],
    out_specs=[_row_spec, _row_spec],
    out_shape=[jax.ShapeDtypeStruct((N, D), jnp.float32)] * 2,
)

_tc2 = pl.pallas_call(
    _tc2_body,
    grid=(N // R,),
    in_specs=[_s_spec, _s_spec, _row_spec, _w_spec, _b_spec],
    out_specs=[_row_spec, _row_spec],
    out_shape=[jax.ShapeDtypeStruct((N, D), jnp.float32)] * 2,
)

_tc3 = pl.pallas_call(
    _tc3_body,
    grid=(N // R,),
    in_specs=[_s_spec, _s_spec, _row_spec, _b_spec],
    out_specs=_row_spec,
    out_shape=jax.ShapeDtypeStruct((N, D), jnp.float32),
)


def kernel(x, adj, W1, b1, W2, b2):
    src = adj[0].astype(jnp.int32)
    dst = adj[1].astype(jnp.int32)
    pad = EP - E
    src_p = jnp.concatenate([src, jnp.zeros((pad,), jnp.int32)])
    dst_p = jnp.concatenate([dst, jnp.full((pad,), N, jnp.int32)])
    src3 = src_p.reshape(NS, SCHUNK, CHUNK)
    dst3 = dst_p.reshape(NS, SCHUNK, CHUNK)

    dp = _deg_kernel(dst3)
    g1, t1 = _tc1(x, W1, dp)
    s1 = _scatter_kernel(g1, src3, dst3)
    t2, g2 = _tc2(s1, dp, t1, W2, b1.reshape(1, D))
    s2 = _scatter_kernel(g2, src3, dst3)
    out = _tc3(s2, dp, t2, b2.reshape(1, D))
    return out


# speedup vs baseline: 6.4064x; 1.1912x over previous
import functools

import jax
import jax.numpy as jnp
from jax import lax
from jax.experimental import pallas as pl
from jax.experimental.pallas import tpu as pltpu
from jax.experimental.pallas import tpu_sc as plsc

N = 10000
E = 320000
D = 128

NC = 2
NS = 16
NW = NC * NS
CHUNK = 128
EP = 325632
SCHUNK = EP // (NS * CHUNK)
NH = 5000
NHPAD = 5120
SRPT = NHPAD // NS

_mesh = plsc.VectorSubcoreMesh(core_axis_name="c", subcore_axis_name="s")


def _zero_slice(buf, acc, base, nrows):
    for k in range(nrows // CHUNK):
        pltpu.sync_copy(buf, acc.at[pl.ds(base + k * CHUNK, CHUNK)])
    rem = nrows % CHUNK
    if rem:
        pltpu.sync_copy(buf.at[pl.ds(0, rem)],
                        acc.at[pl.ds(base + nrows - rem, rem)])


@functools.partial(
    pl.kernel,
    out_type=jax.ShapeDtypeStruct((NC, NHPAD, D), jnp.float32),
    mesh=_mesh,
    scratch_types=[
        pltpu.VMEM((SCHUNK, CHUNK), jnp.int32),
        pltpu.VMEM((CHUNK, D), jnp.float32),
        pltpu.VMEM_SHARED((NHPAD, D), jnp.float32),
    ],
)
def _deg_kernel(dst3, out, dst_v, buf, dacc):
    c = lax.axis_index("c")
    s = lax.axis_index("s")
    pltpu.sync_copy(dst3.at[s], dst_v)

    def fill(val):
        def row(i, carry):
            for k in range(D // 16):
                buf[i, pl.ds(k * 16, 16)] = jnp.full((16,), val, jnp.float32)
            return carry
        lax.fori_loop(0, CHUNK, row, 0)

    fill(0.0)
    base = s * SRPT
    _zero_slice(buf, dacc, base, SRPT)

    lo = c * NH

    def remap(j, carry):
        for k in range(CHUNK // 16):
            v = dst_v[j, pl.ds(k * 16, 16)]
            lc = v - lo
            ok = (lc >= 0) & (lc < NH)
            dst_v[j, pl.ds(k * 16, 16)] = jnp.where(ok, lc, NH)
        return carry
    lax.fori_loop(0, SCHUNK, remap, 0)

    fill(1.0)
    plsc.subcore_barrier()

    def chunk(j, carry):
        pltpu.sync_copy(buf, dacc.at[dst_v.at[j]], add=True)
        return carry
    lax.fori_loop(0, SCHUNK, chunk, 0)
    plsc.subcore_barrier()
    pltpu.sync_copy(dacc.at[pl.ds(base, SRPT)], out.at[c, pl.ds(base, SRPT)])


@functools.partial(
    pl.kernel,
    out_type=jax.ShapeDtypeStruct((NC, NHPAD, D), jnp.float32),
    mesh=_mesh,
    scratch_types=[
        pltpu.VMEM((SCHUNK, CHUNK), jnp.int32),
        pltpu.VMEM((SCHUNK, CHUNK), jnp.int32),
        [pltpu.VMEM((CHUNK, D), jnp.float32)] * 3,
        [pltpu.SemaphoreType.DMA] * 3,
        [pltpu.SemaphoreType.DMA] * 3,
        pltpu.VMEM_SHARED((NHPAD, D), jnp.float32),
    ],
)
def _scatter_kernel(g, src3, dst3, out, src_v, dst_v, bufs, gsem, ssem, acc):
    c = lax.axis_index("c")
    s = lax.axis_index("s")
    pltpu.sync_copy(src3.at[s], src_v)
    pltpu.sync_copy(dst3.at[s], dst_v)
    def zrow(i, carry):
        for k in range(D // 16):
            bufs[0][i, pl.ds(k * 16, 16)] = jnp.zeros((16,), jnp.float32)
        return carry
    lax.fori_loop(0, CHUNK, zrow, 0)
    base = s * SRPT
    _zero_slice(bufs[0], acc, base, SRPT)

    lo = c * NH

    def remap(j, carry):
        for k in range(CHUNK // 16):
            v = dst_v[j, pl.ds(k * 16, 16)]
            lc = v - lo
            ok = (lc >= 0) & (lc < NH)
            dst_v[j, pl.ds(k * 16, 16)] = jnp.where(ok, lc, NH)
        return carry
    lax.fori_loop(0, SCHUNK, remap, 0)
    plsc.subcore_barrier()

    def fire(j, b):
        pltpu.async_copy(g.at[src_v.at[j]], bufs[b], gsem[b])

    def wait_g(b):
        pltpu.make_async_copy(g.at[pl.ds(0, CHUNK)], bufs[b], gsem[b]).wait()

    def wait_s(b):
        pltpu.make_async_copy(bufs[b], acc.at[dst_v.at[0]], ssem[b]).wait()

    fire(0, 0)
    fire(1, 1)

    def ring(i, carry):
        for k in range(3):
            j = 3 * i + k
            bn = (k + 2) % 3

            @pl.when(j >= 1)
            def _():
                wait_s(bn)

            @pl.when(j + 2 < SCHUNK)
            def _():
                fire(j + 2, bn)

            wait_g(k)
            pltpu.async_copy(bufs[k], acc.at[dst_v.at[j]], ssem[k], add=True)
        return carry

    lax.fori_loop(0, SCHUNK // 3, ring, 0)
    wait_s((SCHUNK - 1) % 3)
    plsc.subcore_barrier()
    pltpu.sync_copy(acc.at[pl.ds(base, SRPT)], out.at[c, pl.ds(base, SRPT)])


R = 1000


def _dinv_of(dp_ref):
    deg = dp_ref[0, :, 0] + 1.0
    return lax.rsqrt(deg)


def _tc1_body(x_ref, w_ref, dp_ref, g_ref, t_ref):
    dinv = _dinv_of(dp_ref)
    t = jnp.dot(x_ref[...], w_ref[...], preferred_element_type=jnp.float32)
    t_ref[...] = t
    g_ref[...] = t * dinv[:, None]


def _tc2_body(s_ref, dp_ref, t1_ref, w_ref, b_ref, t2_ref, g2_ref):
    dinv = _dinv_of(dp_ref)
    h = (s_ref[0] * dinv[:, None]
         + t1_ref[...] * (dinv * dinv)[:, None] + b_ref[...])
    t2 = jnp.dot(h, w_ref[...], preferred_element_type=jnp.float32)
    t2_ref[...] = t2
    g2_ref[...] = t2 * dinv[:, None]


def _tc3_body(s_ref, dp_ref, t2_ref, b_ref, out_ref):
    dinv = _dinv_of(dp_ref)
    out_ref[...] = (s_ref[0] * dinv[:, None]
                    + t2_ref[...] * (dinv * dinv)[:, None] + b_ref[...])


_row_spec = pl.BlockSpec((R, D), lambda i: (i, 0))
_w_spec = pl.BlockSpec((D, D), lambda i: (0, 0))
_s_spec = pl.BlockSpec((1, R, D), lambda i: (i // (NH // R), i % (NH // R), 0))
_b_spec = pl.BlockSpec((1, D), lambda i: (0, 0))

_tc1 = pl.pallas_call(
    _tc1_body,
    grid=(N // R,),
    in_specs=[_row_spec, _w_spec, _s_spec],
    out_specs=[_row_spec, _row_spec],
    out_shape=[jax.ShapeDtypeStruct((N, D), jnp.float32)] * 2,
)

_tc2 = pl.pallas_call(
    _tc2_body,
    grid=(N // R,),
    in_specs=[_s_spec, _s_spec, _row_spec, _w_spec, _b_spec],
    out_specs=[_row_spec, _row_spec],
    out_shape=[jax.ShapeDtypeStruct((N, D), jnp.float32)] * 2,
)

_tc3 = pl.pallas_call(
    _tc3_body,
    grid=(N // R,),
    in_specs=[_s_spec, _s_spec, _row_spec, _b_spec],
    out_specs=_row_spec,
    out_shape=jax.ShapeDtypeStruct((N, D), jnp.float32),
)


def kernel(x, adj, W1, b1, W2, b2):
    src = adj[0].astype(jnp.int32)
    dst = adj[1].astype(jnp.int32)
    pad = EP - E
    src_p = jnp.concatenate([src, jnp.zeros((pad,), jnp.int32)])
    dst_p = jnp.concatenate([dst, jnp.full((pad,), N, jnp.int32)])
    src3 = src_p.reshape(NS, SCHUNK, CHUNK)
    dst3 = dst_p.reshape(NS, SCHUNK, CHUNK)

    dp = _deg_kernel(dst3)
    g1, t1 = _tc1(x, W1, dp)
    s1 = _scatter_kernel(g1, src3, dst3)
    t2, g2 = _tc2(s1, dp, t1, W2, b1.reshape(1, D))
    s2 = _scatter_kernel(g2, src3, dst3)
    out = _tc3(s2, dp, t2, b2.reshape(1, D))
    return out
